# baseline (speedup 1.0000x reference)
import functools

import jax
import jax.numpy as jnp
from jax import lax
from jax.experimental import pallas as pl
from jax.experimental.pallas import tpu as pltpu
from jax.experimental.pallas import tpu_sc as plsc

NUM_CORES = 2
NUM_SUBCORES = 16
NUM_WORKERS = NUM_CORES * NUM_SUBCORES
CHUNK = 128
NBUF = 5
LAG = 2
LOOKAHEAD = NBUF - LAG


@functools.partial(jax.jit, static_argnums=(2, 3))
def _gather(ids2d, weight, b, d):
    b_per_w = b // NUM_WORKERS
    nchunks = b_per_w // CHUNK
    assert nchunks % NBUF == 0 and nchunks >= 2 * NBUF
    nouter = nchunks // NBUF
    mesh = plsc.VectorSubcoreMesh(core_axis_name="c", subcore_axis_name="s")

    @functools.partial(
        pl.kernel,
        out_type=jax.ShapeDtypeStruct((b, d), jnp.float32),
        mesh=mesh,
        scratch_types=[
            pltpu.VMEM((nchunks, CHUNK), jnp.int32),
            pltpu.VMEM((NBUF, CHUNK, d), jnp.float32),
            pltpu.SemaphoreType.DMA,
            pltpu.SemaphoreType.DMA,
        ],
    )
    def k(idx_hbm, table_hbm, out_hbm, idx_v, rows_v, gsem, ssem):
        wid = lax.axis_index("s") * NUM_CORES + lax.axis_index("c")
        cbase = wid * nchunks
        rbase = wid * b_per_w

        pltpu.sync_copy(idx_hbm.at[pl.ds(cbase, nchunks)], idx_v)

        def start_gather(buf, j):
            pltpu.async_copy(table_hbm.at[idx_v.at[j]], rows_v.at[buf], gsem)

        def wait_gather(buf):
            pltpu.make_async_copy(
                table_hbm.at[idx_v.at[0]], rows_v.at[buf], gsem
            ).wait()

        def start_store(buf, j):
            pltpu.async_copy(
                rows_v.at[buf], out_hbm.at[pl.ds(rbase + j * CHUNK, CHUNK)], ssem
            )

        def wait_store_one(buf):
            pltpu.make_async_copy(
                rows_v.at[buf], out_hbm.at[pl.ds(rbase, CHUNK)], ssem
            ).wait()

        def step(b, j, first_round, last_round):
            buf = b % NBUF
            wait_gather(buf)
            start_store(buf, j)
            if not (first_round and b < LAG):
                wait_store_one((b - LAG) % NBUF)
            if not (last_round and b >= NBUF - LOOKAHEAD):
                start_gather((b + LOOKAHEAD) % NBUF, j + LOOKAHEAD)

        for buf in range(LOOKAHEAD):
            start_gather(buf, buf)

        for b in range(NBUF):
            step(b, b, True, False)

        @pl.loop(1, nouter - 1)
        def _(t):
            for b in range(NBUF):
                step(b, t * NBUF + b, False, False)

        for b in range(NBUF):
            step(b, (nouter - 1) * NBUF + b, False, True)

        for b in range(LAG):
            wait_store_one((NBUF - LAG + b) % NBUF)

    return k(ids2d, weight)


def kernel(token_ids, weight):
    s, t = token_ids.shape
    n, d = weight.shape
    b = s * t
    ids2d = token_ids.reshape(b // CHUNK, CHUNK).astype(jnp.int32)
    out = _gather(ids2d, weight, b, d)
    return out.reshape(s, t, d)


# speedup vs baseline: 1.0002x; 1.0002x over previous
import functools

import jax
import jax.numpy as jnp
from jax import lax
from jax.experimental import pallas as pl
from jax.experimental.pallas import tpu as pltpu
from jax.experimental.pallas import tpu_sc as plsc

NUM_CORES = 2
NUM_SUBCORES = 16
NUM_WORKERS = NUM_CORES * NUM_SUBCORES
CHUNK = 128
NBUF = 5
LAG = 1
LOOKAHEAD = NBUF - LAG


@functools.partial(jax.jit, static_argnums=(2, 3))
def _gather(ids2d, weight, b, d):
    b_per_w = b // NUM_WORKERS
    nchunks = b_per_w // CHUNK
    assert nchunks % NBUF == 0 and nchunks >= 2 * NBUF
    nouter = nchunks // NBUF
    mesh = plsc.VectorSubcoreMesh(core_axis_name="c", subcore_axis_name="s")

    @functools.partial(
        pl.kernel,
        out_type=jax.ShapeDtypeStruct((b, d), jnp.float32),
        mesh=mesh,
        scratch_types=[
            pltpu.VMEM((nchunks, CHUNK), jnp.int32),
            pltpu.VMEM((NBUF, CHUNK, d), jnp.float32),
            pltpu.SemaphoreType.DMA,
            pltpu.SemaphoreType.DMA,
        ],
    )
    def k(idx_hbm, table_hbm, out_hbm, idx_v, rows_v, gsem, ssem):
        wid = lax.axis_index("s") * NUM_CORES + lax.axis_index("c")
        cbase = wid * nchunks
        rbase = wid * b_per_w

        pltpu.sync_copy(idx_hbm.at[pl.ds(cbase, nchunks)], idx_v)

        def start_gather(buf, j):
            pltpu.async_copy(table_hbm.at[idx_v.at[j]], rows_v.at[buf], gsem)

        def wait_gather(buf):
            pltpu.make_async_copy(
                table_hbm.at[idx_v.at[0]], rows_v.at[buf], gsem
            ).wait()

        def start_store(buf, j):
            pltpu.async_copy(
                rows_v.at[buf], out_hbm.at[pl.ds(rbase + j * CHUNK, CHUNK)], ssem
            )

        def wait_store_one(buf):
            pltpu.make_async_copy(
                rows_v.at[buf], out_hbm.at[pl.ds(rbase, CHUNK)], ssem
            ).wait()

        def step(b, j, first_round, last_round):
            buf = b % NBUF
            wait_gather(buf)
            start_store(buf, j)
            if not (first_round and b < LAG):
                wait_store_one((b - LAG) % NBUF)
            if not (last_round and b >= NBUF - LOOKAHEAD):
                start_gather((b + LOOKAHEAD) % NBUF, j + LOOKAHEAD)

        for buf in range(LOOKAHEAD):
            start_gather(buf, buf)

        for b in range(NBUF):
            step(b, b, True, False)

        @pl.loop(1, nouter - 1)
        def _(t):
            for b in range(NBUF):
                step(b, t * NBUF + b, False, False)

        for b in range(NBUF):
            step(b, (nouter - 1) * NBUF + b, False, True)

        for b in range(LAG):
            wait_store_one((NBUF - LAG + b) % NBUF)

    return k(ids2d, weight)


def kernel(token_ids, weight):
    s, t = token_ids.shape
    n, d = weight.shape
    b = s * t
    ids2d = token_ids.reshape(b // CHUNK, CHUNK).astype(jnp.int32)
    out = _gather(ids2d, weight, b, d)
    return out.reshape(s, t, d)
